# baseline (device time: 95512 ns/iter reference)
import jax
import jax.numpy as jnp
from jax import lax
from jax.experimental import pallas as pl
from jax.experimental.pallas import tpu as pltpu

KQ = 16
KF = KQ // 2
KS = 3
LAG = 2


def kernel(x):
    m, n = x.shape
    Q = m // 4
    rows = Q // KQ
    halfq = Q // 2

    def body(x_hbm, out_hbm, xq_ref, comm_ref, gx_ref, gy_ref,
             z_send, z_recv, xa_send, ya_send,
             gx_recv, gy_recv,
             xoA_recv, xoB_recv, xoC_recv,
             yoA_recv, yoB_recv, yoC_recv,
             fx_send, fy_send, fxo_recv, fyo_recv,
             zx_send, zx_recv, zy_send, zy_recv,
             lin_sem, lred_sem, lgx_sem, lgy_sem):
        my_x = lax.axis_index("x")
        my_y = lax.axis_index("y")
        my_z = lax.axis_index("z")
        z_nbr = (my_x, my_y, 1 - my_z)
        x_nbr = (1 - my_x, my_y, my_z)
        y_nbr = (my_x, 1 - my_y, my_z)
        q_me = 2 * my_x + my_y
        q_x = 2 * (1 - my_x) + my_y
        q_y = 2 * my_x + (1 - my_y)
        q_d = 2 * (1 - my_x) + (1 - my_y)
        b_me = q_me * Q
        is_z0 = my_z == 0
        is_z1 = my_z == 1

        barrier_sem = pltpu.get_barrier_semaphore()
        for nbr in (z_nbr, x_nbr, y_nbr):
            pl.semaphore_signal(
                barrier_sem, inc=1,
                device_id=nbr, device_id_type=pl.DeviceIdType.MESH,
            )
        pl.semaphore_wait(barrier_sem, 3)

        def rdma(src, dst, ssem, rsem, dev):
            return pltpu.make_async_remote_copy(
                src_ref=src, dst_ref=dst, send_sem=ssem, recv_sem=rsem,
                device_id=dev, device_id_type=pl.DeviceIdType.MESH,
            )

        def when_start(pred, desc):
            @pl.when(pred)
            def _():
                desc.start()

        def when_wait_send(pred, desc):
            @pl.when(pred)
            def _():
                desc.wait_send()

        def when_wait_recv(pred, desc):
            @pl.when(pred)
            def _():
                desc.wait_recv()

        def y_alpha_pred_sem(k):
            if k < KS:
                return is_z1, yoA_recv.at[k]
            if k < 2 * KS:
                return is_z0, yoB_recv.at[k - KS]
            return None, yoC_recv.at[k - 2 * KS]

        def x_beta_pred_sem(k):
            j = k - KF
            if j < KS:
                return is_z1, xoA_recv.at[j]
            if j < 2 * KS:
                return is_z0, xoB_recv.at[j - KS]
            return None, xoC_recv.at[j - 2 * KS]

        z_rdmas, local_in = [], []
        for k in range(KQ):
            src_sl = pl.ds(b_me + k * rows, rows)
            dst_sl = pl.ds(k * rows, rows)
            r = rdma(x_hbm.at[src_sl, :], comm_ref.at[dst_sl, :],
                     z_send.at[k], z_recv.at[k], z_nbr)
            r.start()
            z_rdmas.append(r)
            c = pltpu.make_async_copy(
                x_hbm.at[src_sl, :], xq_ref.at[dst_sl, :], lin_sem.at[k]
            )
            c.start()
            local_in.append(c)

        fy_rdmas, local_gx = [], []

        def fy_forward(j):
            sl = pl.ds(j * rows, rows)
            out_sl = pl.ds(q_x * Q + j * rows, rows)
            rdma(gx_ref.at[sl, :], gx_ref.at[sl, :],
                 xa_send.at[j], gx_recv.at[j], x_nbr).wait_recv()
            r = rdma(gx_ref.at[sl, :], out_hbm.at[out_sl, :],
                     fy_send.at[j], fyo_recv.at[j], y_nbr)
            r.start()
            fy_rdmas.append(r)
            c = pltpu.make_async_copy(
                gx_ref.at[sl, :], out_hbm.at[out_sl, :], lgx_sem.at[j]
            )
            c.start()
            local_gx.append(c)

        fx_rdmas, local_gy = [], []

        def fx_forward(j):
            sl = pl.ds(j * rows, rows)
            out_sl = pl.ds(q_y * Q + halfq + j * rows, rows)
            rdma(gy_ref.at[sl, :], gy_ref.at[sl, :],
                 ya_send.at[KF + j], gy_recv.at[j], y_nbr).wait_recv()
            r = rdma(gy_ref.at[sl, :], out_hbm.at[out_sl, :],
                     fx_send.at[j], fxo_recv.at[j], x_nbr)
            r.start()
            fx_rdmas.append(r)
            c = pltpu.make_async_copy(
                gy_ref.at[sl, :], out_hbm.at[out_sl, :], lgy_sem.at[j]
            )
            c.start()
            local_gy.append(c)

        zx_rdmas, zy_rdmas = [], []

        def zy_share(j):
            h1_sl = pl.ds(q_y * Q + j * rows, rows)
            h2_sl = pl.ds(q_y * Q + KS * rows + j * rows, rows)
            when_wait_recv(is_z0, rdma(out_hbm.at[h2_sl, :], out_hbm.at[h2_sl, :],
                                       ya_send.at[0], yoB_recv.at[j], y_nbr))
            when_wait_recv(is_z1, rdma(out_hbm.at[h1_sl, :], out_hbm.at[h1_sl, :],
                                       ya_send.at[0], yoA_recv.at[j], y_nbr))
            r0 = rdma(out_hbm.at[h2_sl, :], out_hbm.at[h2_sl, :],
                      zy_send.at[j], zy_recv.at[j], z_nbr)
            r1 = rdma(out_hbm.at[h1_sl, :], out_hbm.at[h1_sl, :],
                      zy_send.at[j], zy_recv.at[j], z_nbr)
            when_start(is_z0, r0)
            when_start(is_z1, r1)
            zy_rdmas.append((r0, r1))

        def zx_share(j):
            g1_sl = pl.ds(q_x * Q + KF * rows + j * rows, rows)
            g2_sl = pl.ds(q_x * Q + (KF + KS) * rows + j * rows, rows)
            when_wait_recv(is_z0, rdma(out_hbm.at[g2_sl, :], out_hbm.at[g2_sl, :],
                                       xa_send.at[0], xoB_recv.at[j], x_nbr))
            when_wait_recv(is_z1, rdma(out_hbm.at[g1_sl, :], out_hbm.at[g1_sl, :],
                                       xa_send.at[0], xoA_recv.at[j], x_nbr))
            r0 = rdma(out_hbm.at[g2_sl, :], out_hbm.at[g2_sl, :],
                      zx_send.at[j], zx_recv.at[j], z_nbr)
            r1 = rdma(out_hbm.at[g1_sl, :], out_hbm.at[g1_sl, :],
                      zx_send.at[j], zx_recv.at[j], z_nbr)
            when_start(is_z0, r0)
            when_start(is_z1, r1)
            zx_rdmas.append((r0, r1))

        xa_list, ya_list, local_red = [], [], []
        for k in range(KQ):
            local_in[k].wait()
            z_rdmas[k].wait_recv()
            sl = pl.ds(k * rows, rows)
            comm_ref[sl, :] = xq_ref[sl, :] + comm_ref[sl, :]
            out_sl = pl.ds(b_me + k * rows, rows)
            c = pltpu.make_async_copy(
                comm_ref.at[sl, :], out_hbm.at[out_sl, :], lred_sem.at[k]
            )
            c.start()
            local_red.append(c)
            if k < KF:
                rx = rdma(comm_ref.at[sl, :], gx_ref.at[sl, :],
                          xa_send.at[k], gx_recv.at[k], x_nbr)
                rx.start()
                xa_list.append((None, rx))
                pred, rsem = y_alpha_pred_sem(k)
                ry = rdma(comm_ref.at[sl, :], out_hbm.at[out_sl, :],
                          ya_send.at[k], rsem, y_nbr)
                if pred is None:
                    ry.start()
                else:
                    when_start(pred, ry)
                ya_list.append((pred, ry))
            else:
                j = k - KF
                pred, rsem = x_beta_pred_sem(k)
                rx = rdma(comm_ref.at[sl, :], out_hbm.at[out_sl, :],
                          xa_send.at[k], rsem, x_nbr)
                if pred is None:
                    rx.start()
                else:
                    when_start(pred, rx)
                xa_list.append((pred, rx))
                ry = rdma(comm_ref.at[sl, :],
                          gy_ref.at[pl.ds(j * rows, rows), :],
                          ya_send.at[k], gy_recv.at[j], y_nbr)
                ry.start()
                ya_list.append((None, ry))

            kk = k - LAG
            if 0 <= kk < KF:
                fy_forward(kk)
            if 0 <= kk - KF < KF:
                fx_forward(kk - KF)
            if 4 + LAG <= k < 4 + LAG + KS:
                zy_share(k - 4 - LAG)
            if 12 + LAG <= k < 12 + LAG + KS:
                zx_share(k - 12 - LAG)

        for j in range(KF - LAG, KF):
            fx_forward(j)
        for j in range(KQ - 12 - LAG, KS):
            zx_share(j)

        for j in range(2):
            sl = pl.ds(q_x * Q + (KF + 2 * KS) * rows + j * rows, rows)
            rdma(out_hbm.at[sl, :], out_hbm.at[sl, :],
                 xa_send.at[0], xoC_recv.at[j], x_nbr).wait_recv()
            sl = pl.ds(q_y * Q + 2 * KS * rows + j * rows, rows)
            rdma(out_hbm.at[sl, :], out_hbm.at[sl, :],
                 ya_send.at[0], yoC_recv.at[j], y_nbr).wait_recv()
        for j in range(KS):
            g1_sl = pl.ds(q_x * Q + KF * rows + j * rows, rows)
            g2_sl = pl.ds(q_x * Q + (KF + KS) * rows + j * rows, rows)
            when_wait_recv(is_z0, rdma(out_hbm.at[g1_sl, :], out_hbm.at[g1_sl, :],
                                       zx_send.at[j], zx_recv.at[j], z_nbr))
            when_wait_recv(is_z1, rdma(out_hbm.at[g2_sl, :], out_hbm.at[g2_sl, :],
                                       zx_send.at[j], zx_recv.at[j], z_nbr))
            h1_sl = pl.ds(q_y * Q + j * rows, rows)
            h2_sl = pl.ds(q_y * Q + KS * rows + j * rows, rows)
            when_wait_recv(is_z0, rdma(out_hbm.at[h1_sl, :], out_hbm.at[h1_sl, :],
                                       zy_send.at[j], zy_recv.at[j], z_nbr))
            when_wait_recv(is_z1, rdma(out_hbm.at[h2_sl, :], out_hbm.at[h2_sl, :],
                                       zy_send.at[j], zy_recv.at[j], z_nbr))
        for j in range(KF):
            sl = pl.ds(q_d * Q + halfq + j * rows, rows)
            rdma(out_hbm.at[sl, :], out_hbm.at[sl, :],
                 fx_send.at[j], fxo_recv.at[j], x_nbr).wait_recv()
            sl = pl.ds(q_d * Q + j * rows, rows)
            rdma(out_hbm.at[sl, :], out_hbm.at[sl, :],
                 fy_send.at[j], fyo_recv.at[j], y_nbr).wait_recv()

        for k in range(KQ):
            z_rdmas[k].wait_send()
            local_red[k].wait()
        for pred, r in xa_list + ya_list:
            if pred is None:
                r.wait_send()
            else:
                when_wait_send(pred, r)
        for j in range(KF):
            fx_rdmas[j].wait_send()
            fy_rdmas[j].wait_send()
            local_gx[j].wait()
            local_gy[j].wait()
        for r0, r1 in zx_rdmas + zy_rdmas:
            when_wait_send(is_z0, r0)
            when_wait_send(is_z1, r1)

    return pl.pallas_call(
        body,
        out_shape=jax.ShapeDtypeStruct((m, n), jnp.float32),
        in_specs=[pl.BlockSpec(memory_space=pl.ANY)],
        out_specs=pl.BlockSpec(memory_space=pl.ANY),
        scratch_shapes=[
            pltpu.VMEM((Q, n), jnp.float32),
            pltpu.VMEM((Q, n), jnp.float32),
            pltpu.VMEM((halfq, n), jnp.float32),
            pltpu.VMEM((halfq, n), jnp.float32),
            pltpu.SemaphoreType.DMA((KQ,)),
            pltpu.SemaphoreType.DMA((KQ,)),
            pltpu.SemaphoreType.DMA((KQ,)),
            pltpu.SemaphoreType.DMA((KQ,)),
            pltpu.SemaphoreType.DMA((KF,)),
            pltpu.SemaphoreType.DMA((KF,)),
            pltpu.SemaphoreType.DMA((KS,)),
            pltpu.SemaphoreType.DMA((KS,)),
            pltpu.SemaphoreType.DMA((2,)),
            pltpu.SemaphoreType.DMA((KS,)),
            pltpu.SemaphoreType.DMA((KS,)),
            pltpu.SemaphoreType.DMA((2,)),
            pltpu.SemaphoreType.DMA((KF,)),
            pltpu.SemaphoreType.DMA((KF,)),
            pltpu.SemaphoreType.DMA((KF,)),
            pltpu.SemaphoreType.DMA((KF,)),
            pltpu.SemaphoreType.DMA((KS,)),
            pltpu.SemaphoreType.DMA((KS,)),
            pltpu.SemaphoreType.DMA((KS,)),
            pltpu.SemaphoreType.DMA((KS,)),
            pltpu.SemaphoreType.DMA((KQ,)),
            pltpu.SemaphoreType.DMA((KQ,)),
            pltpu.SemaphoreType.DMA((KF,)),
            pltpu.SemaphoreType.DMA((KF,)),
        ],
        compiler_params=pltpu.CompilerParams(collective_id=0),
    )(x)
